# Initial kernel scaffold; baseline (speedup 1.0000x reference)
import jax
import jax.numpy as jnp
from jax.experimental import pallas as pl


def kernel(rec_x, rec_edge_attr, rec_coord, lig_x, lig_edge_attr, lig_coord, pre_rot, trans, params, rec_edge_index, lig_edge_index):
    raise NotImplementedError("write your pallas kernel here")


# speedup vs baseline: 1.9379x; 1.9379x over previous
import functools

import jax
import jax.numpy as jnp
from jax import lax
from jax.experimental import pallas as pl
from jax.experimental.pallas import tpu as pltpu
from jax.experimental.pallas import tpu_sc as plsc

N_REC = 10000
E_REC = 160000
N_LIG = 256
E_LIG = 1024
D_FEAT = 128
D_EDGE = 16
OUT = 64
NLAYERS = 3
NT = 4
GRAD_COEF = 0.1

F32 = jnp.float32

NC = 2
NS = 16
NW = NC * NS
CHUNK = 128
CHUNKS_PER_W = 40
EPW = CHUNK * CHUNKS_PER_W
E_PAD = EPW * NW
AGG_ROWS = 10240
ROWS_PER_TILE = AGG_ROWS // NS

BE = 8000
BN = 2000
RB = 2048
R_PAD = 10240


def _ew_body(ea_ref, we_ref, be_ref, out_ref):
    out_ref[0] = (
        jnp.dot(ea_ref[...], we_ref[0], preferred_element_type=F32) + be_ref[0]
    )


def _compute_ew(edge_attr, we_bot, be):
    out = pl.pallas_call(
        _ew_body,
        grid=(NLAYERS, E_REC // BE),
        in_specs=[
            pl.BlockSpec((BE, D_EDGE), lambda l, e: (e, 0)),
            pl.BlockSpec((1, D_EDGE, OUT), lambda l, e: (l, 0, 0)),
            pl.BlockSpec((1, 1, OUT), lambda l, e: (l, 0, 0)),
        ],
        out_specs=pl.BlockSpec((1, BE, OUT), lambda l, e: (l, e, 0)),
        out_shape=jax.ShapeDtypeStruct((NLAYERS, E_PAD, OUT), F32),
    )(edge_attr, we_bot, be)
    return out.reshape(NLAYERS * E_PAD, OUT)


def _rec_in_body(x_ref, win_ref, bin_ref, wet_ref, h_ref, hw_ref):
    h = jnp.maximum(
        jnp.dot(x_ref[...], win_ref[...], preferred_element_type=F32)
        + bin_ref[...],
        0.0,
    )
    h_ref[...] = h
    hw_ref[...] = jnp.dot(h, wet_ref[...], preferred_element_type=F32)


def _rec_input(rec_x, win, bin_, wet0):
    return pl.pallas_call(
        _rec_in_body,
        grid=(N_REC // BN,),
        in_specs=[
            pl.BlockSpec((BN, D_FEAT), lambda i: (i, 0)),
            pl.BlockSpec((D_FEAT, OUT), lambda i: (0, 0)),
            pl.BlockSpec((1, OUT), lambda i: (0, 0)),
            pl.BlockSpec((OUT, OUT), lambda i: (0, 0)),
        ],
        out_specs=[
            pl.BlockSpec((BN, OUT), lambda i: (i, 0)),
            pl.BlockSpec((BN, OUT), lambda i: (i, 0)),
        ],
        out_shape=[
            jax.ShapeDtypeStruct((N_REC, OUT), F32),
            jax.ShapeDtypeStruct((N_REC, OUT), F32),
        ],
    )(rec_x, win, bin_, wet0)


def _make_sc_edge(layer):
    mesh = plsc.VectorSubcoreMesh(
        core_axis_name="c", subcore_axis_name="s",
        num_cores=NC, num_subcores=NS,
    )

    @functools.partial(
        pl.kernel,
        out_type=jax.ShapeDtypeStruct((NC, AGG_ROWS, OUT), F32),
        mesh=mesh,
        scratch_types=[
            pltpu.VMEM((CHUNK,), jnp.int32),
            pltpu.VMEM((CHUNK,), jnp.int32),
            pltpu.VMEM((CHUNK, OUT), F32),
            pltpu.VMEM((CHUNK, OUT), F32),
            pltpu.VMEM_SHARED((AGG_ROWS, OUT), F32),
            pltpu.SemaphoreType.DMA,
        ],
        compiler_params=pltpu.CompilerParams(use_tc_tiling_on_sc=False),
    )
    def sc_edge(hw_hbm, src_hbm, dst_hbm, ew_hbm, out_hbm,
                src_v, dst_v, gat_v, msg_v, agg_sh, sem):
        cid = lax.axis_index("c")
        sid = lax.axis_index("s")
        wid = sid * NC + cid

        def _zrow(r, carry):
            for cc in range(OUT // 16):
                msg_v[r, pl.ds(cc * 16, 16)] = jnp.zeros((16,), F32)
            return carry

        lax.fori_loop(0, CHUNK, _zrow, 0)
        for k2 in range(ROWS_PER_TILE // CHUNK):
            pltpu.sync_copy(
                msg_v,
                agg_sh.at[pl.ds(sid * ROWS_PER_TILE + k2 * CHUNK, CHUNK)],
            )
        plsc.subcore_barrier()

        def _chunk(j, carry):
            base = wid * EPW + j * CHUNK
            pltpu.sync_copy(src_hbm.at[pl.ds(base, CHUNK)], src_v)
            pltpu.sync_copy(dst_hbm.at[pl.ds(base, CHUNK)], dst_v)
            pltpu.sync_copy(
                ew_hbm.at[pl.ds(layer * E_PAD + base, CHUNK)], msg_v
            )
            pltpu.async_copy(hw_hbm.at[src_v], gat_v, sem).wait()

            def _row(r, c2):
                for cc in range(OUT // 16):
                    sl = pl.ds(cc * 16, 16)
                    msg_v[r, sl] = jnp.maximum(msg_v[r, sl] + gat_v[r, sl], 0.0)
                return c2

            lax.fori_loop(0, CHUNK, _row, 0)
            pltpu.sync_copy(msg_v, agg_sh.at[dst_v], add=True)
            return carry

        lax.fori_loop(0, CHUNKS_PER_W, _chunk, 0)
        plsc.subcore_barrier()

        for k2 in range(ROWS_PER_TILE // CHUNK):
            r0 = sid * ROWS_PER_TILE + k2 * CHUNK
            pltpu.sync_copy(
                agg_sh.at[pl.ds(r0, CHUNK)], out_hbm.at[cid, pl.ds(r0, CHUNK)]
            )

    return sc_edge


@functools.lru_cache(maxsize=None)
def _sc_edge_cached(layer):
    return _make_sc_edge(layer)


def _sc_edge_call(layer, hw, src, dst, ew):
    return _sc_edge_cached(layer)(hw, src, dst, ew)


def _node_body(h_ref, agg_ref, wnt_ref, wnb_ref, bn_ref, wet_ref,
               h_out, hw_out):
    a = agg_ref[0] + agg_ref[1]
    hn = jnp.maximum(
        jnp.dot(h_ref[...], wnt_ref[...], preferred_element_type=F32)
        + jnp.dot(a, wnb_ref[...], preferred_element_type=F32)
        + bn_ref[...],
        0.0,
    )
    h_out[...] = hn
    hw_out[...] = jnp.dot(hn, wet_ref[...], preferred_element_type=F32)


def _node_body_last(h_ref, agg_ref, wnt_ref, wnb_ref, bn_ref, h_out):
    a = agg_ref[0] + agg_ref[1]
    h_out[...] = jnp.maximum(
        jnp.dot(h_ref[...], wnt_ref[...], preferred_element_type=F32)
        + jnp.dot(a, wnb_ref[...], preferred_element_type=F32)
        + bn_ref[...],
        0.0,
    )


def _node_update(h, agg2, wnt, wnb, bn, wet_next):
    last = wet_next is None
    in_specs = [
        pl.BlockSpec((BN, OUT), lambda i: (i, 0)),
        pl.BlockSpec((NC, BN, OUT), lambda i: (0, i, 0)),
        pl.BlockSpec((OUT, OUT), lambda i: (0, 0)),
        pl.BlockSpec((OUT, OUT), lambda i: (0, 0)),
        pl.BlockSpec((1, OUT), lambda i: (0, 0)),
    ]
    args = [h, agg2, wnt, wnb, bn]
    if last:
        return pl.pallas_call(
            _node_body_last,
            grid=(N_REC // BN,),
            in_specs=in_specs,
            out_specs=pl.BlockSpec((BN, OUT), lambda i: (i, 0)),
            out_shape=jax.ShapeDtypeStruct((N_REC, OUT), F32),
        )(*args)
    in_specs.append(pl.BlockSpec((OUT, OUT), lambda i: (0, 0)))
    args.append(wet_next)
    return pl.pallas_call(
        _node_body,
        grid=(N_REC // BN,),
        in_specs=in_specs,
        out_specs=[
            pl.BlockSpec((BN, OUT), lambda i: (i, 0)),
            pl.BlockSpec((BN, OUT), lambda i: (i, 0)),
        ],
        out_shape=[
            jax.ShapeDtypeStruct((N_REC, OUT), F32),
            jax.ShapeDtypeStruct((N_REC, OUT), F32),
        ],
    )(*args)


def _lig_body(x_ref, ea_ref, src_ref, dst_ref, win_ref, bin_ref,
              wet_ref, web_ref, be_ref, wnt_ref, wnb_ref, bn_ref, out_ref):
    iot = lax.broadcasted_iota(jnp.int32, (E_LIG, N_LIG), 1)
    ohs = (src_ref[...] == iot).astype(F32)
    ohd = (dst_ref[...] == iot).astype(F32)
    h = jnp.maximum(
        jnp.dot(x_ref[...], win_ref[...], preferred_element_type=F32)
        + bin_ref[...],
        0.0,
    )
    for l in range(NLAYERS):
        hsrc = jnp.dot(ohs, h, preferred_element_type=F32)
        m = jnp.maximum(
            jnp.dot(hsrc, wet_ref[l], preferred_element_type=F32)
            + jnp.dot(ea_ref[...], web_ref[l], preferred_element_type=F32)
            + be_ref[l],
            0.0,
        )
        agg = lax.dot_general(
            ohd, m, (((0,), (0,)), ((), ())), preferred_element_type=F32
        )
        h = jnp.maximum(
            jnp.dot(h, wnt_ref[l], preferred_element_type=F32)
            + jnp.dot(agg, wnb_ref[l], preferred_element_type=F32)
            + bn_ref[l],
            0.0,
        )
    out_ref[...] = h


def _lig_mpnn(lig_x, lig_ea, src2, dst2, win, bin_, wet, web, be, wnt, wnb, bn):
    return pl.pallas_call(
        _lig_body,
        out_shape=jax.ShapeDtypeStruct((N_LIG, OUT), F32),
    )(lig_x, lig_ea, src2, dst2, win, bin_, wet, web, be, wnt, wnb, bn)


def _energy_body(lf_ref, rf_ref, rc_ref, pos_ref, f_ref):
    i = pl.program_id(0)

    @pl.when(i == 0)
    def _():
        f_ref[...] = jnp.zeros_like(f_ref)

    atn = lax.dot_general(
        lf_ref[...], rf_ref[...], (((1,), (1,)), ((), ())),
        preferred_element_type=F32,
    )
    rcx = rc_ref[0, :][None, :]
    rcy = rc_ref[1, :][None, :]
    rcz = rc_ref[2, :][None, :]
    for t in range(NT):
        dx = pos_ref[3 * t + 0, :][:, None] - rcx
        dy = pos_ref[3 * t + 1, :][:, None] - rcy
        dz = pos_ref[3 * t + 2, :][:, None] - rcz
        d2 = dx * dx + dy * dy + dz * dz + 1e-12
        inv = lax.rsqrt(d2)
        w = atn * (inv * inv * inv)
        f_ref[3 * t + 0, :] += -jnp.sum(w * dx, axis=1)
        f_ref[3 * t + 1, :] += -jnp.sum(w * dy, axis=1)
        f_ref[3 * t + 2, :] += -jnp.sum(w * dz, axis=1)


def _forces(lig_feat, rec_feat_pad, rec_c_t, pos_flat):
    return pl.pallas_call(
        _energy_body,
        grid=(R_PAD // RB,),
        in_specs=[
            pl.BlockSpec((N_LIG, OUT), lambda i: (0, 0)),
            pl.BlockSpec((RB, OUT), lambda i: (i, 0)),
            pl.BlockSpec((3, RB), lambda i: (0, i)),
            pl.BlockSpec((3 * NT, N_LIG), lambda i: (0, 0)),
        ],
        out_specs=pl.BlockSpec((3 * NT, N_LIG), lambda i: (0, 0)),
        out_shape=jax.ShapeDtypeStruct((3 * NT, N_LIG), F32),
    )(lig_feat, rec_feat_pad, rec_c_t, pos_flat)


def _gs_q(A):
    a0 = A[..., :, 0]
    a1 = A[..., :, 1]
    a2 = A[..., :, 2]

    def _norm(v):
        return v / jnp.sqrt((v * v).sum(-1, keepdims=True) + 1e-12)

    q0 = _norm(a0)
    u1 = a1 - (a1 * q0).sum(-1, keepdims=True) * q0
    q1 = _norm(u1)
    u2 = a2 - (a2 * q0).sum(-1, keepdims=True) * q0 - (a2 * q1).sum(-1, keepdims=True) * q1
    q2 = _norm(u2)
    return jnp.stack([q0, q1, q2], axis=-1)


def kernel(rec_x, rec_edge_attr, rec_coord, lig_x, lig_edge_attr, lig_coord,
           pre_rot, trans, params, rec_edge_index, lig_edge_index):
    pr, pl_ = params["rec"], params["lig"]

    def _split(p):
        wet = jnp.stack([p[f"W_e{l}"][:OUT] for l in range(NLAYERS)])
        web = jnp.stack([p[f"W_e{l}"][OUT:] for l in range(NLAYERS)])
        be = jnp.stack([p[f"b_e{l}"][None] for l in range(NLAYERS)])
        wnt = jnp.stack([p[f"W_n{l}"][:OUT] for l in range(NLAYERS)])
        wnb = jnp.stack([p[f"W_n{l}"][OUT:] for l in range(NLAYERS)])
        bn = jnp.stack([p[f"b_n{l}"][None] for l in range(NLAYERS)])
        return wet, web, be, wnt, wnb, bn

    r_wet, r_web, r_be, r_wnt, r_wnb, r_bn = _split(pr)
    l_wet, l_web, l_be, l_wnt, l_wnb, l_bn = _split(pl_)

    ew = _compute_ew(rec_edge_attr, r_web, r_be)
    h, hw = _rec_input(rec_x, pr["W_in"], pr["b_in"][None], r_wet[0])

    src = jnp.pad(rec_edge_index[0], (0, E_PAD - E_REC))
    dst = jnp.pad(rec_edge_index[1], (0, E_PAD - E_REC),
                  constant_values=N_REC)

    for l in range(NLAYERS):
        agg2 = _sc_edge_call(l, hw, src, dst, ew)
        wet_next = r_wet[l + 1] if l + 1 < NLAYERS else None
        res = _node_update(h, agg2, r_wnt[l], r_wnb[l], r_bn[l], wet_next)
        if wet_next is None:
            h = res
        else:
            h, hw = res
    rec_feat = h

    lig_feat = _lig_mpnn(
        lig_x, lig_edge_attr,
        lig_edge_index[0][:, None], lig_edge_index[1][:, None],
        pl_["W_in"], pl_["b_in"][None],
        l_wet, l_web, l_be, l_wnt, l_wnb, l_bn,
    )

    lig_c = lig_coord - lig_coord.mean(0)
    rec_c = rec_coord - rec_coord.mean(0)
    rot = _gs_q(pre_rot)
    pos = jnp.einsum("tij,lj->til", rot, lig_c) + trans[:, :, None]
    rec_feat_pad = jnp.pad(rec_feat, ((0, R_PAD - N_REC), (0, 0)))
    rec_c_t = jnp.pad(rec_c.T, ((0, 0), (0, R_PAD - N_REC)))

    F = _forces(lig_feat, rec_feat_pad, rec_c_t,
                pos.reshape(3 * NT, N_LIG))
    F = F.reshape(NT, 3, N_LIG)

    g_rot_rot = jnp.einsum("til,lj->tij", F, lig_c) / NT
    g_trans = F.sum(-1) / NT

    _, vjp_fn = jax.vjp(_gs_q, pre_rot)
    (g_pre,) = vjp_fn(g_rot_rot)

    final_rot = _gs_q(pre_rot - GRAD_COEF * g_pre)
    final_trans = trans - GRAD_COEF * g_trans
    return final_rot, final_trans


# speedup vs baseline: 2.5423x; 1.3119x over previous
import functools

import jax
import jax.numpy as jnp
from jax import lax
from jax.experimental import pallas as pl
from jax.experimental.pallas import tpu as pltpu
from jax.experimental.pallas import tpu_sc as plsc

N_REC = 10000
E_REC = 160000
N_LIG = 256
E_LIG = 1024
D_FEAT = 128
D_EDGE = 16
OUT = 64
NLAYERS = 3
NT = 4
GRAD_COEF = 0.1

F32 = jnp.float32

NC = 2
NS = 16
NW = NC * NS
CHUNK = 128
CHUNKS_PER_W = 40
EPW = CHUNK * CHUNKS_PER_W
E_PAD = EPW * NW
AGG_ROWS = 10240
ROWS_PER_TILE = AGG_ROWS // NS

BE = 8000
BN = 2000
RB = 2048
R_PAD = 10240


def _ew_body(ea_ref, we_ref, be_ref, out_ref):
    out_ref[0] = (
        jnp.dot(ea_ref[...], we_ref[0], preferred_element_type=F32) + be_ref[0]
    )


def _compute_ew(edge_attr, we_bot, be):
    return pl.pallas_call(
        _ew_body,
        grid=(NLAYERS, E_REC // BE),
        in_specs=[
            pl.BlockSpec((BE, D_EDGE), lambda l, e: (e, 0)),
            pl.BlockSpec((1, D_EDGE, OUT), lambda l, e: (l, 0, 0)),
            pl.BlockSpec((1, 1, OUT), lambda l, e: (l, 0, 0)),
        ],
        out_specs=pl.BlockSpec((1, BE, OUT), lambda l, e: (l, e, 0)),
        out_shape=jax.ShapeDtypeStruct((NLAYERS, E_PAD, OUT), F32),
    )(edge_attr, we_bot, be)


def _rec_in_body(x_ref, win_ref, bin_ref, wet_ref, h_ref, hw_ref):
    h = jnp.maximum(
        jnp.dot(x_ref[...], win_ref[...], preferred_element_type=F32)
        + bin_ref[...],
        0.0,
    )
    h_ref[...] = h
    hw_ref[...] = jnp.dot(h, wet_ref[...], preferred_element_type=F32)


def _rec_input(rec_x, win, bin_, wet0):
    return pl.pallas_call(
        _rec_in_body,
        grid=(N_REC // BN,),
        in_specs=[
            pl.BlockSpec((BN, D_FEAT), lambda i: (i, 0)),
            pl.BlockSpec((D_FEAT, OUT), lambda i: (0, 0)),
            pl.BlockSpec((1, OUT), lambda i: (0, 0)),
            pl.BlockSpec((OUT, OUT), lambda i: (0, 0)),
        ],
        out_specs=[
            pl.BlockSpec((BN, OUT), lambda i: (i, 0)),
            pl.BlockSpec((BN, OUT), lambda i: (i, 0)),
        ],
        out_shape=[
            jax.ShapeDtypeStruct((N_REC, OUT), F32),
            jax.ShapeDtypeStruct((N_REC, OUT), F32),
        ],
    )(rec_x, win, bin_, wet0)


NBUF = 5
OUTER = CHUNKS_PER_W // NBUF


def _make_sc_edge(layer):
    mesh = plsc.VectorSubcoreMesh(
        core_axis_name="c", subcore_axis_name="s",
        num_cores=NC, num_subcores=NS,
    )

    @functools.partial(
        pl.kernel,
        out_type=jax.ShapeDtypeStruct((NC, AGG_ROWS, OUT), F32),
        mesh=mesh,
        scratch_types=[
            [pltpu.VMEM((2, CHUNK), jnp.int32) for _ in range(NBUF)],
            [pltpu.VMEM((CHUNK, OUT), F32) for _ in range(NBUF)],
            [pltpu.VMEM((CHUNK, OUT), F32) for _ in range(NBUF)],
            pltpu.VMEM_SHARED((AGG_ROWS, OUT), F32),
            pltpu.SemaphoreType.DMA,
            pltpu.SemaphoreType.DMA,
            pltpu.SemaphoreType.DMA,
        ],
        compiler_params=pltpu.CompilerParams(use_tc_tiling_on_sc=False),
    )
    def sc_edge(hw_hbm, idx_hbm, ew_hbm, out_hbm,
                idx_v, gat_v, msg_v, agg_sh, lsem, gsem, ssem):
        cid = lax.axis_index("c")
        sid = lax.axis_index("s")
        wid = sid * NC + cid

        @plsc.parallel_loop(0, CHUNK, unroll=8)
        def _zrow(r):
            for cc in range(OUT // 16):
                msg_v[0][r, pl.ds(cc * 16, 16)] = jnp.zeros((16,), F32)

        for k2 in range(ROWS_PER_TILE // CHUNK):
            pltpu.sync_copy(
                msg_v[0],
                agg_sh.at[pl.ds(sid * ROWS_PER_TILE + k2 * CHUNK, CHUNK)],
            )
        plsc.subcore_barrier()

        def _outer(g, carry):
            cb0 = wid * CHUNKS_PER_W + g * NBUF
            ld = []
            for b in range(NBUF):
                base = (cb0 + b) * CHUNK
                ld.append((
                    pltpu.async_copy(idx_hbm.at[cb0 + b], idx_v[b], lsem),
                    pltpu.async_copy(
                        ew_hbm.at[layer, pl.ds(base, CHUNK)], msg_v[b], lsem
                    ),
                ))
            gd = []
            for b in range(NBUF):
                ld[b][0].wait()
                ld[b][1].wait()
                gd.append(
                    pltpu.async_copy(hw_hbm.at[idx_v[b].at[0]], gat_v[b], gsem)
                )
            sd = []
            for b in range(NBUF):
                gd[b].wait()

                @plsc.parallel_loop(0, CHUNK, unroll=4)
                def _row(r):
                    for cc in range(OUT // 16):
                        sl = pl.ds(cc * 16, 16)
                        msg_v[b][r, sl] = jnp.maximum(
                            msg_v[b][r, sl] + gat_v[b][r, sl], 0.0
                        )

                sd.append(
                    pltpu.async_copy(
                        msg_v[b], agg_sh.at[idx_v[b].at[1]], ssem, add=True
                    )
                )
            for b in range(NBUF):
                sd[b].wait()
            return carry

        lax.fori_loop(0, OUTER, _outer, 0)
        plsc.subcore_barrier()

        for k2 in range(ROWS_PER_TILE // CHUNK):
            r0 = sid * ROWS_PER_TILE + k2 * CHUNK
            pltpu.sync_copy(
                agg_sh.at[pl.ds(r0, CHUNK)], out_hbm.at[cid, pl.ds(r0, CHUNK)]
            )

    return sc_edge


@functools.lru_cache(maxsize=None)
def _sc_edge_cached(layer):
    return _make_sc_edge(layer)


def _sc_edge_call(layer, hw, idx_packed, ew):
    return _sc_edge_cached(layer)(hw, idx_packed, ew)


def _node_body(h_ref, agg_ref, wnt_ref, wnb_ref, bn_ref, wet_ref,
               h_out, hw_out):
    a = agg_ref[0] + agg_ref[1]
    hn = jnp.maximum(
        jnp.dot(h_ref[...], wnt_ref[...], preferred_element_type=F32)
        + jnp.dot(a, wnb_ref[...], preferred_element_type=F32)
        + bn_ref[...],
        0.0,
    )
    h_out[...] = hn
    hw_out[...] = jnp.dot(hn, wet_ref[...], preferred_element_type=F32)


def _node_body_last(h_ref, agg_ref, wnt_ref, wnb_ref, bn_ref, h_out):
    a = agg_ref[0] + agg_ref[1]
    h_out[...] = jnp.maximum(
        jnp.dot(h_ref[...], wnt_ref[...], preferred_element_type=F32)
        + jnp.dot(a, wnb_ref[...], preferred_element_type=F32)
        + bn_ref[...],
        0.0,
    )


def _node_update(h, agg2, wnt, wnb, bn, wet_next):
    last = wet_next is None
    in_specs = [
        pl.BlockSpec((BN, OUT), lambda i: (i, 0)),
        pl.BlockSpec((NC, BN, OUT), lambda i: (0, i, 0)),
        pl.BlockSpec((OUT, OUT), lambda i: (0, 0)),
        pl.BlockSpec((OUT, OUT), lambda i: (0, 0)),
        pl.BlockSpec((1, OUT), lambda i: (0, 0)),
    ]
    args = [h, agg2, wnt, wnb, bn]
    if last:
        return pl.pallas_call(
            _node_body_last,
            grid=(N_REC // BN,),
            in_specs=in_specs,
            out_specs=pl.BlockSpec((BN, OUT), lambda i: (i, 0)),
            out_shape=jax.ShapeDtypeStruct((N_REC, OUT), F32),
        )(*args)
    in_specs.append(pl.BlockSpec((OUT, OUT), lambda i: (0, 0)))
    args.append(wet_next)
    return pl.pallas_call(
        _node_body,
        grid=(N_REC // BN,),
        in_specs=in_specs,
        out_specs=[
            pl.BlockSpec((BN, OUT), lambda i: (i, 0)),
            pl.BlockSpec((BN, OUT), lambda i: (i, 0)),
        ],
        out_shape=[
            jax.ShapeDtypeStruct((N_REC, OUT), F32),
            jax.ShapeDtypeStruct((N_REC, OUT), F32),
        ],
    )(*args)


def _lig_body(x_ref, ea_ref, src_ref, dst_ref, win_ref, bin_ref,
              wet_ref, web_ref, be_ref, wnt_ref, wnb_ref, bn_ref, out_ref):
    iot = lax.broadcasted_iota(jnp.int32, (E_LIG, N_LIG), 1)
    ohs = (src_ref[...] == iot).astype(F32)
    ohd = (dst_ref[...] == iot).astype(F32)
    h = jnp.maximum(
        jnp.dot(x_ref[...], win_ref[...], preferred_element_type=F32)
        + bin_ref[...],
        0.0,
    )
    for l in range(NLAYERS):
        hsrc = jnp.dot(ohs, h, preferred_element_type=F32)
        m = jnp.maximum(
            jnp.dot(hsrc, wet_ref[l], preferred_element_type=F32)
            + jnp.dot(ea_ref[...], web_ref[l], preferred_element_type=F32)
            + be_ref[l],
            0.0,
        )
        agg = lax.dot_general(
            ohd, m, (((0,), (0,)), ((), ())), preferred_element_type=F32
        )
        h = jnp.maximum(
            jnp.dot(h, wnt_ref[l], preferred_element_type=F32)
            + jnp.dot(agg, wnb_ref[l], preferred_element_type=F32)
            + bn_ref[l],
            0.0,
        )
    out_ref[...] = h


def _lig_mpnn(lig_x, lig_ea, src2, dst2, win, bin_, wet, web, be, wnt, wnb, bn):
    return pl.pallas_call(
        _lig_body,
        out_shape=jax.ShapeDtypeStruct((N_LIG, OUT), F32),
    )(lig_x, lig_ea, src2, dst2, win, bin_, wet, web, be, wnt, wnb, bn)


def _energy_body(lf_ref, rf_ref, rc_ref, pos_ref, f_ref):
    i = pl.program_id(0)

    @pl.when(i == 0)
    def _():
        f_ref[...] = jnp.zeros_like(f_ref)

    atn = lax.dot_general(
        lf_ref[...], rf_ref[...], (((1,), (1,)), ((), ())),
        preferred_element_type=F32,
    )
    rcx = rc_ref[0, :][None, :]
    rcy = rc_ref[1, :][None, :]
    rcz = rc_ref[2, :][None, :]
    for t in range(NT):
        dx = pos_ref[3 * t + 0, :][:, None] - rcx
        dy = pos_ref[3 * t + 1, :][:, None] - rcy
        dz = pos_ref[3 * t + 2, :][:, None] - rcz
        d2 = dx * dx + dy * dy + dz * dz + 1e-12
        inv = lax.rsqrt(d2)
        w = atn * (inv * inv * inv)
        f_ref[3 * t + 0, :] += -jnp.sum(w * dx, axis=1)
        f_ref[3 * t + 1, :] += -jnp.sum(w * dy, axis=1)
        f_ref[3 * t + 2, :] += -jnp.sum(w * dz, axis=1)


def _forces(lig_feat, rec_feat_pad, rec_c_t, pos_flat):
    return pl.pallas_call(
        _energy_body,
        grid=(R_PAD // RB,),
        in_specs=[
            pl.BlockSpec((N_LIG, OUT), lambda i: (0, 0)),
            pl.BlockSpec((RB, OUT), lambda i: (i, 0)),
            pl.BlockSpec((3, RB), lambda i: (0, i)),
            pl.BlockSpec((3 * NT, N_LIG), lambda i: (0, 0)),
        ],
        out_specs=pl.BlockSpec((3 * NT, N_LIG), lambda i: (0, 0)),
        out_shape=jax.ShapeDtypeStruct((3 * NT, N_LIG), F32),
    )(lig_feat, rec_feat_pad, rec_c_t, pos_flat)


def _gs_q(A):
    a0 = A[..., :, 0]
    a1 = A[..., :, 1]
    a2 = A[..., :, 2]

    def _norm(v):
        return v / jnp.sqrt((v * v).sum(-1, keepdims=True) + 1e-12)

    q0 = _norm(a0)
    u1 = a1 - (a1 * q0).sum(-1, keepdims=True) * q0
    q1 = _norm(u1)
    u2 = a2 - (a2 * q0).sum(-1, keepdims=True) * q0 - (a2 * q1).sum(-1, keepdims=True) * q1
    q2 = _norm(u2)
    return jnp.stack([q0, q1, q2], axis=-1)


def kernel(rec_x, rec_edge_attr, rec_coord, lig_x, lig_edge_attr, lig_coord,
           pre_rot, trans, params, rec_edge_index, lig_edge_index):
    pr, pl_ = params["rec"], params["lig"]

    def _split(p):
        wet = jnp.stack([p[f"W_e{l}"][:OUT] for l in range(NLAYERS)])
        web = jnp.stack([p[f"W_e{l}"][OUT:] for l in range(NLAYERS)])
        be = jnp.stack([p[f"b_e{l}"][None] for l in range(NLAYERS)])
        wnt = jnp.stack([p[f"W_n{l}"][:OUT] for l in range(NLAYERS)])
        wnb = jnp.stack([p[f"W_n{l}"][OUT:] for l in range(NLAYERS)])
        bn = jnp.stack([p[f"b_n{l}"][None] for l in range(NLAYERS)])
        return wet, web, be, wnt, wnb, bn

    r_wet, r_web, r_be, r_wnt, r_wnb, r_bn = _split(pr)
    l_wet, l_web, l_be, l_wnt, l_wnb, l_bn = _split(pl_)

    ew = _compute_ew(rec_edge_attr, r_web, r_be)
    h, hw = _rec_input(rec_x, pr["W_in"], pr["b_in"][None], r_wet[0])

    src = jnp.pad(rec_edge_index[0], (0, E_PAD - E_REC))
    dst = jnp.pad(rec_edge_index[1], (0, E_PAD - E_REC),
                  constant_values=N_REC)
    idx_packed = jnp.stack(
        [src.reshape(-1, CHUNK), dst.reshape(-1, CHUNK)], axis=1
    )

    for l in range(NLAYERS):
        agg2 = _sc_edge_call(l, hw, idx_packed, ew)
        wet_next = r_wet[l + 1] if l + 1 < NLAYERS else None
        res = _node_update(h, agg2, r_wnt[l], r_wnb[l], r_bn[l], wet_next)
        if wet_next is None:
            h = res
        else:
            h, hw = res
    rec_feat = h

    lig_feat = _lig_mpnn(
        lig_x, lig_edge_attr,
        lig_edge_index[0][:, None], lig_edge_index[1][:, None],
        pl_["W_in"], pl_["b_in"][None],
        l_wet, l_web, l_be, l_wnt, l_wnb, l_bn,
    )

    lig_c = lig_coord - lig_coord.mean(0)
    rec_c = rec_coord - rec_coord.mean(0)
    rot = _gs_q(pre_rot)
    pos = jnp.einsum("tij,lj->til", rot, lig_c) + trans[:, :, None]
    rec_feat_pad = jnp.pad(rec_feat, ((0, R_PAD - N_REC), (0, 0)))
    rec_c_t = jnp.pad(rec_c.T, ((0, 0), (0, R_PAD - N_REC)))

    F = _forces(lig_feat, rec_feat_pad, rec_c_t,
                pos.reshape(3 * NT, N_LIG))
    F = F.reshape(NT, 3, N_LIG)

    g_rot_rot = jnp.einsum("til,lj->tij", F, lig_c) / NT
    g_trans = F.sum(-1) / NT

    _, vjp_fn = jax.vjp(_gs_q, pre_rot)
    (g_pre,) = vjp_fn(g_rot_rot)

    final_rot = _gs_q(pre_rot - GRAD_COEF * g_pre)
    final_trans = trans - GRAD_COEF * g_trans
    return final_rot, final_trans


# speedup vs baseline: 2.6954x; 1.0602x over previous
import functools

import jax
import jax.numpy as jnp
from jax import lax
from jax.experimental import pallas as pl
from jax.experimental.pallas import tpu as pltpu
from jax.experimental.pallas import tpu_sc as plsc

N_REC = 10000
E_REC = 160000
N_LIG = 256
E_LIG = 1024
D_FEAT = 128
D_EDGE = 16
OUT = 64
NLAYERS = 3
NT = 4
GRAD_COEF = 0.1

F32 = jnp.float32

NC = 2
NS = 16
NW = NC * NS
CHUNK = 128
CHUNKS_PER_W = 40
EPW = CHUNK * CHUNKS_PER_W
E_PAD = EPW * NW
AGG_ROWS = 10240
ROWS_PER_TILE = AGG_ROWS // NS

HWIDE = 64

BE = 8000
BN = 2000
RB = 2048
R_PAD = 10240


def _ew_body(ea_ref, we_ref, be_ref, out_ref):
    for l in range(NLAYERS):
        out_ref[l] = (
            jnp.dot(ea_ref[...], we_ref[l], preferred_element_type=F32)
            + be_ref[l]
        )


def _compute_ew(edge_attr, we_bot, be):
    return pl.pallas_call(
        _ew_body,
        grid=(E_REC // BE,),
        in_specs=[
            pl.BlockSpec((BE, D_EDGE), lambda e: (e, 0)),
            pl.BlockSpec((NLAYERS, D_EDGE, OUT), lambda e: (0, 0, 0)),
            pl.BlockSpec((NLAYERS, 1, OUT), lambda e: (0, 0, 0)),
        ],
        out_specs=pl.BlockSpec((NLAYERS, BE, OUT), lambda e: (0, e, 0)),
        out_shape=jax.ShapeDtypeStruct((NLAYERS, E_PAD, OUT), F32),
    )(edge_attr, we_bot, be)


def _rec_in_body(x_ref, win_ref, bin_ref, wet_ref, h_ref, hw_ref):
    h = jnp.maximum(
        jnp.dot(x_ref[...], win_ref[...], preferred_element_type=F32)
        + bin_ref[...],
        0.0,
    )
    h_ref[...] = h
    hw_ref[...] = jnp.dot(h, wet_ref[...], preferred_element_type=F32)


def _rec_input(rec_x, win, bin_, wet0):
    return pl.pallas_call(
        _rec_in_body,
        grid=(N_REC // BN,),
        in_specs=[
            pl.BlockSpec((BN, D_FEAT), lambda i: (i, 0)),
            pl.BlockSpec((D_FEAT, OUT), lambda i: (0, 0)),
            pl.BlockSpec((1, OUT), lambda i: (0, 0)),
            pl.BlockSpec((OUT, HWIDE), lambda i: (0, 0)),
        ],
        out_specs=[
            pl.BlockSpec((BN, OUT), lambda i: (i, 0)),
            pl.BlockSpec((BN, HWIDE), lambda i: (i, 0)),
        ],
        out_shape=[
            jax.ShapeDtypeStruct((N_REC, OUT), F32),
            jax.ShapeDtypeStruct((N_REC, HWIDE), F32),
        ],
    )(rec_x, win, bin_, wet0)


NBUF = 5
OUTER = CHUNKS_PER_W // NBUF


def _make_sc_edge(layer):
    mesh = plsc.VectorSubcoreMesh(
        core_axis_name="c", subcore_axis_name="s",
        num_cores=NC, num_subcores=NS,
    )

    @functools.partial(
        pl.kernel,
        out_type=jax.ShapeDtypeStruct((NC, AGG_ROWS, OUT), F32),
        mesh=mesh,
        scratch_types=[
            [pltpu.VMEM((2, CHUNK), jnp.int32) for _ in range(NBUF)],
            [pltpu.VMEM((CHUNK, HWIDE), F32) for _ in range(NBUF)],
            [pltpu.VMEM((CHUNK, OUT), F32) for _ in range(NBUF)],
            pltpu.VMEM_SHARED((AGG_ROWS, OUT), F32),
            pltpu.SemaphoreType.DMA,
            pltpu.SemaphoreType.DMA,
            pltpu.SemaphoreType.DMA,
        ],
        compiler_params=pltpu.CompilerParams(use_tc_tiling_on_sc=False),
    )
    def sc_edge(hw_hbm, idx_hbm, ew_hbm, out_hbm,
                idx_v, gat_v, msg_v, agg_sh, lsem, gsem, ssem):
        cid = lax.axis_index("c")
        sid = lax.axis_index("s")
        wid = sid * NC + cid

        @plsc.parallel_loop(0, CHUNK, unroll=8)
        def _zrow(r):
            for cc in range(OUT // 16):
                msg_v[0][r, pl.ds(cc * 16, 16)] = jnp.zeros((16,), F32)

        for k2 in range(ROWS_PER_TILE // CHUNK):
            pltpu.sync_copy(
                msg_v[0],
                agg_sh.at[pl.ds(sid * ROWS_PER_TILE + k2 * CHUNK, CHUNK)],
            )
        plsc.subcore_barrier()

        def _outer(g, carry):
            cb0 = wid * CHUNKS_PER_W + g * NBUF
            ld = []
            for b in range(NBUF):
                base = (cb0 + b) * CHUNK
                ld.append((
                    pltpu.async_copy(
                        idx_hbm.at[pl.ds(2 * (cb0 + b), 2)], idx_v[b], lsem
                    ),
                    pltpu.async_copy(
                        ew_hbm.at[layer, pl.ds(base, CHUNK)], msg_v[b], lsem
                    ),
                ))
            gd = []
            for b in range(NBUF):
                ld[b][0].wait()
                ld[b][1].wait()
                gd.append(
                    pltpu.async_copy(hw_hbm.at[idx_v[b].at[0]], gat_v[b], gsem)
                )
            sd = []
            for b in range(NBUF):
                gd[b].wait()

                @plsc.parallel_loop(0, CHUNK, unroll=4)
                def _row(r):
                    for cc in range(OUT // 16):
                        sl = pl.ds(cc * 16, 16)
                        msg_v[b][r, sl] = jnp.maximum(
                            msg_v[b][r, sl] + gat_v[b][r, sl], 0.0
                        )

                sd.append(
                    pltpu.async_copy(
                        msg_v[b], agg_sh.at[idx_v[b].at[1]], ssem, add=True
                    )
                )
            for b in range(NBUF):
                sd[b].wait()
            return carry

        lax.fori_loop(0, OUTER, _outer, 0)
        plsc.subcore_barrier()

        for k2 in range(ROWS_PER_TILE // CHUNK):
            r0 = sid * ROWS_PER_TILE + k2 * CHUNK
            pltpu.sync_copy(
                agg_sh.at[pl.ds(r0, CHUNK)], out_hbm.at[cid, pl.ds(r0, CHUNK)]
            )

    return sc_edge


@functools.lru_cache(maxsize=None)
def _sc_edge_cached(layer):
    return _make_sc_edge(layer)


def _sc_edge_call(layer, hw, idx_packed, ew):
    return _sc_edge_cached(layer)(hw, idx_packed, ew)


def _node_body(h_ref, agg_ref, wnt_ref, wnb_ref, bn_ref, wet_ref,
               h_out, hw_out):
    a = agg_ref[0] + agg_ref[1]
    hn = jnp.maximum(
        jnp.dot(h_ref[...], wnt_ref[...], preferred_element_type=F32)
        + jnp.dot(a, wnb_ref[...], preferred_element_type=F32)
        + bn_ref[...],
        0.0,
    )
    h_out[...] = hn
    hw_out[...] = jnp.dot(hn, wet_ref[...], preferred_element_type=F32)


def _node_body_last(h_ref, agg_ref, wnt_ref, wnb_ref, bn_ref, h_out):
    a = agg_ref[0] + agg_ref[1]
    h_out[...] = jnp.maximum(
        jnp.dot(h_ref[...], wnt_ref[...], preferred_element_type=F32)
        + jnp.dot(a, wnb_ref[...], preferred_element_type=F32)
        + bn_ref[...],
        0.0,
    )


def _node_update(h, agg2, wnt, wnb, bn, wet_next):
    last = wet_next is None
    in_specs = [
        pl.BlockSpec((BN, OUT), lambda i: (i, 0)),
        pl.BlockSpec((NC, BN, OUT), lambda i: (0, i, 0)),
        pl.BlockSpec((OUT, OUT), lambda i: (0, 0)),
        pl.BlockSpec((OUT, OUT), lambda i: (0, 0)),
        pl.BlockSpec((1, OUT), lambda i: (0, 0)),
    ]
    args = [h, agg2, wnt, wnb, bn]
    if last:
        return pl.pallas_call(
            _node_body_last,
            grid=(N_REC // BN,),
            in_specs=in_specs,
            out_specs=pl.BlockSpec((BN, OUT), lambda i: (i, 0)),
            out_shape=jax.ShapeDtypeStruct((N_REC, OUT), F32),
        )(*args)
    in_specs.append(pl.BlockSpec((OUT, HWIDE), lambda i: (0, 0)))
    args.append(wet_next)
    return pl.pallas_call(
        _node_body,
        grid=(N_REC // BN,),
        in_specs=in_specs,
        out_specs=[
            pl.BlockSpec((BN, OUT), lambda i: (i, 0)),
            pl.BlockSpec((BN, HWIDE), lambda i: (i, 0)),
        ],
        out_shape=[
            jax.ShapeDtypeStruct((N_REC, OUT), F32),
            jax.ShapeDtypeStruct((N_REC, HWIDE), F32),
        ],
    )(*args)


def _lig_body(x_ref, ea_ref, src_ref, dst_ref, win_ref, bin_ref,
              wet_ref, web_ref, be_ref, wnt_ref, wnb_ref, bn_ref, out_ref):
    iot = lax.broadcasted_iota(jnp.int32, (E_LIG, N_LIG), 1)
    ohs = (src_ref[...] == iot).astype(F32)
    ohd = (dst_ref[...] == iot).astype(F32)
    h = jnp.maximum(
        jnp.dot(x_ref[...], win_ref[...], preferred_element_type=F32)
        + bin_ref[...],
        0.0,
    )
    for l in range(NLAYERS):
        hsrc = jnp.dot(ohs, h, preferred_element_type=F32)
        m = jnp.maximum(
            jnp.dot(hsrc, wet_ref[l], preferred_element_type=F32)
            + jnp.dot(ea_ref[...], web_ref[l], preferred_element_type=F32)
            + be_ref[l],
            0.0,
        )
        agg = lax.dot_general(
            ohd, m, (((0,), (0,)), ((), ())), preferred_element_type=F32
        )
        h = jnp.maximum(
            jnp.dot(h, wnt_ref[l], preferred_element_type=F32)
            + jnp.dot(agg, wnb_ref[l], preferred_element_type=F32)
            + bn_ref[l],
            0.0,
        )
    out_ref[...] = h


def _lig_mpnn(lig_x, lig_ea, src2, dst2, win, bin_, wet, web, be, wnt, wnb, bn):
    return pl.pallas_call(
        _lig_body,
        out_shape=jax.ShapeDtypeStruct((N_LIG, OUT), F32),
    )(lig_x, lig_ea, src2, dst2, win, bin_, wet, web, be, wnt, wnb, bn)


def _energy_body(lf_ref, rf_ref, rc_ref, pos_ref, f_ref):
    i = pl.program_id(0)

    @pl.when(i == 0)
    def _():
        f_ref[...] = jnp.zeros_like(f_ref)

    atn = lax.dot_general(
        lf_ref[...], rf_ref[...], (((1,), (1,)), ((), ())),
        preferred_element_type=F32,
    )
    rcx = rc_ref[0, :][None, :]
    rcy = rc_ref[1, :][None, :]
    rcz = rc_ref[2, :][None, :]
    for t in range(NT):
        dx = pos_ref[3 * t + 0, :][:, None] - rcx
        dy = pos_ref[3 * t + 1, :][:, None] - rcy
        dz = pos_ref[3 * t + 2, :][:, None] - rcz
        d2 = dx * dx + dy * dy + dz * dz + 1e-12
        inv = lax.rsqrt(d2)
        w = atn * (inv * inv * inv)
        f_ref[3 * t + 0, :] += -jnp.sum(w * dx, axis=1)
        f_ref[3 * t + 1, :] += -jnp.sum(w * dy, axis=1)
        f_ref[3 * t + 2, :] += -jnp.sum(w * dz, axis=1)


def _forces(lig_feat, rec_feat_pad, rec_c_t, pos_flat):
    return pl.pallas_call(
        _energy_body,
        grid=(R_PAD // RB,),
        in_specs=[
            pl.BlockSpec((N_LIG, OUT), lambda i: (0, 0)),
            pl.BlockSpec((RB, OUT), lambda i: (i, 0)),
            pl.BlockSpec((3, RB), lambda i: (0, i)),
            pl.BlockSpec((3 * NT, N_LIG), lambda i: (0, 0)),
        ],
        out_specs=pl.BlockSpec((3 * NT, N_LIG), lambda i: (0, 0)),
        out_shape=jax.ShapeDtypeStruct((3 * NT, N_LIG), F32),
    )(lig_feat, rec_feat_pad, rec_c_t, pos_flat)


def _gs_q(A):
    a0 = A[..., :, 0]
    a1 = A[..., :, 1]
    a2 = A[..., :, 2]

    def _norm(v):
        return v / jnp.sqrt((v * v).sum(-1, keepdims=True) + 1e-12)

    q0 = _norm(a0)
    u1 = a1 - (a1 * q0).sum(-1, keepdims=True) * q0
    q1 = _norm(u1)
    u2 = a2 - (a2 * q0).sum(-1, keepdims=True) * q0 - (a2 * q1).sum(-1, keepdims=True) * q1
    q2 = _norm(u2)
    return jnp.stack([q0, q1, q2], axis=-1)


def kernel(rec_x, rec_edge_attr, rec_coord, lig_x, lig_edge_attr, lig_coord,
           pre_rot, trans, params, rec_edge_index, lig_edge_index):
    pr, pl_ = params["rec"], params["lig"]

    def _split(p):
        wet = jnp.stack([p[f"W_e{l}"][:OUT] for l in range(NLAYERS)])
        web = jnp.stack([p[f"W_e{l}"][OUT:] for l in range(NLAYERS)])
        be = jnp.stack([p[f"b_e{l}"][None] for l in range(NLAYERS)])
        wnt = jnp.stack([p[f"W_n{l}"][:OUT] for l in range(NLAYERS)])
        wnb = jnp.stack([p[f"W_n{l}"][OUT:] for l in range(NLAYERS)])
        bn = jnp.stack([p[f"b_n{l}"][None] for l in range(NLAYERS)])
        return wet, web, be, wnt, wnb, bn

    r_wet, r_web, r_be, r_wnt, r_wnb, r_bn = _split(pr)
    l_wet, l_web, l_be, l_wnt, l_wnb, l_bn = _split(pl_)

    ew = _compute_ew(rec_edge_attr, r_web, r_be)
    h, hw = _rec_input(rec_x, pr["W_in"], pr["b_in"][None], r_wet[0])

    src = jnp.pad(rec_edge_index[0], (0, E_PAD - E_REC))
    dst = jnp.pad(rec_edge_index[1], (0, E_PAD - E_REC),
                  constant_values=N_REC)
    idx_packed = jnp.stack(
        [src.reshape(-1, CHUNK), dst.reshape(-1, CHUNK)], axis=1
    ).reshape(-1, CHUNK)

    for l in range(NLAYERS):
        agg2 = _sc_edge_call(l, hw, idx_packed, ew)
        wet_next = r_wet[l + 1] if l + 1 < NLAYERS else None
        res = _node_update(h, agg2, r_wnt[l], r_wnb[l], r_bn[l], wet_next)
        if wet_next is None:
            h = res
        else:
            h, hw = res
    rec_feat = h

    lig_feat = _lig_mpnn(
        lig_x, lig_edge_attr,
        lig_edge_index[0][:, None], lig_edge_index[1][:, None],
        pl_["W_in"], pl_["b_in"][None],
        l_wet, l_web, l_be, l_wnt, l_wnb, l_bn,
    )

    lig_c = lig_coord - lig_coord.mean(0)
    rec_c = rec_coord - rec_coord.mean(0)
    rot = _gs_q(pre_rot)
    pos = jnp.einsum("tij,lj->til", rot, lig_c) + trans[:, :, None]
    rec_feat_pad = jnp.pad(rec_feat, ((0, R_PAD - N_REC), (0, 0)))
    rec_c_t = jnp.pad(rec_c.T, ((0, 0), (0, R_PAD - N_REC)))

    F = _forces(lig_feat, rec_feat_pad, rec_c_t,
                pos.reshape(3 * NT, N_LIG))
    F = F.reshape(NT, 3, N_LIG)

    g_rot_rot = jnp.einsum("til,lj->tij", F, lig_c) / NT
    g_trans = F.sum(-1) / NT

    _, vjp_fn = jax.vjp(_gs_q, pre_rot)
    (g_pre,) = vjp_fn(g_rot_rot)

    final_rot = _gs_q(pre_rot - GRAD_COEF * g_pre)
    final_trans = trans - GRAD_COEF * g_trans
    return final_rot, final_trans
